# baseline (speedup 1.0000x reference)
import functools

import jax
import jax.numpy as jnp
from jax import lax
from jax.experimental import pallas as pl
from jax.experimental.pallas import tpu as pltpu
from jax.experimental.pallas import tpu_sc as plsc

NC = 1
NS = 16
NW = NC * NS
L = 16


def _gather16(table, idx):
    dnums = lax.GatherDimensionNumbers(
        offset_dims=(), collapsed_slice_dims=(0,), start_index_map=(0,))
    return lax.gather(table, idx[:, None], dnums, (1,),
                      mode=lax.GatherScatterMode.PROMISE_IN_BOUNDS)


def _scale_shift_body(n, energy_hbm, types_hbm, scales_hbm, shifts_hbm,
                      out_hbm, energy_v, types_v, out_v, scales_v, shifts_v,
                      sem_e, sem_t, sem_e2, sem_t2, sem_o):
    wid = lax.axis_index("s") * NC + lax.axis_index("c")
    chunk = energy_v.shape[0]
    half = chunk // 2
    base = lax.min(wid * chunk, n - chunk)
    cp_e0 = pltpu.async_copy(
        energy_hbm.at[pl.ds(base, half)], energy_v.at[pl.ds(0, half)], sem_e)
    cp_t0 = pltpu.async_copy(
        types_hbm.at[pl.ds(base, half)], types_v.at[pl.ds(0, half)], sem_t)
    cp_e1 = pltpu.async_copy(
        energy_hbm.at[pl.ds(base + half, half)],
        energy_v.at[pl.ds(half, half)], sem_e2)
    cp_t1 = pltpu.async_copy(
        types_hbm.at[pl.ds(base + half, half)],
        types_v.at[pl.ds(half, half)], sem_t2)
    cp_s = pltpu.async_copy(scales_hbm, scales_v, sem_e)
    cp_b = pltpu.async_copy(shifts_hbm, shifts_v, sem_t)
    cp_s.wait()
    cp_b.wait()
    cp_e0.wait()
    cp_t0.wait()
    sv = scales_v[...]
    bv = shifts_v[...]

    @plsc.parallel_loop(0, half, L, unroll=4)
    def _(i):
        t = types_v[pl.ds(i, L)]
        s = _gather16(sv, t)
        b = _gather16(bv, t)
        e = energy_v[pl.ds(i, L)]
        out_v[pl.ds(i, L)] = s * e + b

    cp_o0 = pltpu.async_copy(
        out_v.at[pl.ds(0, half)], out_hbm.at[pl.ds(base, half)], sem_o)
    cp_e1.wait()
    cp_t1.wait()

    @plsc.parallel_loop(half, chunk, L, unroll=4)
    def _(i):
        t = types_v[pl.ds(i, L)]
        s = _gather16(sv, t)
        b = _gather16(bv, t)
        e = energy_v[pl.ds(i, L)]
        out_v[pl.ds(i, L)] = s * e + b

    pltpu.sync_copy(
        out_v.at[pl.ds(half, half)], out_hbm.at[pl.ds(base + half, half)])
    cp_o0.wait()


@functools.lru_cache(maxsize=None)
def _build(n):
    chunk = -(-n // (NW * 2 * L)) * (2 * L)
    mesh = plsc.VectorSubcoreMesh(
        core_axis_name="c", subcore_axis_name="s",
        num_cores=NC, num_subcores=NS)
    return pl.kernel(
        functools.partial(_scale_shift_body, n),
        out_type=jax.ShapeDtypeStruct((n,), jnp.float32),
        mesh=mesh,
        scratch_types=[
            pltpu.VMEM((chunk,), jnp.float32),
            pltpu.VMEM((chunk,), jnp.int32),
            pltpu.VMEM((chunk,), jnp.float32),
            pltpu.VMEM((L,), jnp.float32),
            pltpu.VMEM((L,), jnp.float32),
            pltpu.SemaphoreType.DMA,
            pltpu.SemaphoreType.DMA,
            pltpu.SemaphoreType.DMA,
            pltpu.SemaphoreType.DMA,
            pltpu.SemaphoreType.DMA,
        ],
    )


def kernel(atomic_energy, atom_types, scales, shifts):
    n = atomic_energy.shape[0]
    e = atomic_energy.reshape(-1)
    t = atom_types.astype(jnp.int32)
    out = _build(n)(e, t, scales, shifts)
    return out.reshape(n, 1)


# speedup vs baseline: 1.0027x; 1.0027x over previous
import functools

import jax
import jax.numpy as jnp
from jax import lax
from jax.experimental import pallas as pl
from jax.experimental.pallas import tpu as pltpu
from jax.experimental.pallas import tpu_sc as plsc

NC = 1
NS = 16
NW = NC * NS
L = 16


def _gather16(table, idx):
    dnums = lax.GatherDimensionNumbers(
        offset_dims=(), collapsed_slice_dims=(0,), start_index_map=(0,))
    return lax.gather(table, idx[:, None], dnums, (1,),
                      mode=lax.GatherScatterMode.PROMISE_IN_BOUNDS)


def _scale_shift_body(n, energy_hbm, types_hbm, scales_hbm, shifts_hbm,
                      out_hbm, energy_v, types_v, out_v, scales_v, shifts_v,
                      sem_e, sem_t, sem_o):
    wid = lax.axis_index("s") * NC + lax.axis_index("c")
    chunk = energy_v.shape[0]
    half = chunk // 2
    base = lax.min(wid * chunk, n - chunk)
    cp_e = pltpu.async_copy(energy_hbm.at[pl.ds(base, chunk)], energy_v, sem_e)
    cp_t = pltpu.async_copy(types_hbm.at[pl.ds(base, chunk)], types_v, sem_t)
    cp_s = pltpu.async_copy(scales_hbm, scales_v, sem_e)
    cp_b = pltpu.async_copy(shifts_hbm, shifts_v, sem_t)
    cp_s.wait()
    cp_b.wait()
    cp_e.wait()
    cp_t.wait()
    sv = scales_v[...]
    bv = shifts_v[...]

    @plsc.parallel_loop(0, half, L, unroll=4)
    def _(i):
        t = types_v[pl.ds(i, L)]
        s = _gather16(sv, t)
        b = _gather16(bv, t)
        e = energy_v[pl.ds(i, L)]
        out_v[pl.ds(i, L)] = s * e + b

    cp_o = pltpu.async_copy(
        out_v.at[pl.ds(0, half)], out_hbm.at[pl.ds(base, half)], sem_o)

    @plsc.parallel_loop(half, chunk, L, unroll=4)
    def _(i):
        t = types_v[pl.ds(i, L)]
        s = _gather16(sv, t)
        b = _gather16(bv, t)
        e = energy_v[pl.ds(i, L)]
        out_v[pl.ds(i, L)] = s * e + b

    pltpu.sync_copy(
        out_v.at[pl.ds(half, half)], out_hbm.at[pl.ds(base + half, half)])
    cp_o.wait()


@functools.lru_cache(maxsize=None)
def _build(n):
    chunk = -(-n // (NW * 2 * L)) * (2 * L)
    mesh = plsc.VectorSubcoreMesh(
        core_axis_name="c", subcore_axis_name="s",
        num_cores=NC, num_subcores=NS)
    return pl.kernel(
        functools.partial(_scale_shift_body, n),
        out_type=jax.ShapeDtypeStruct((n,), jnp.float32),
        mesh=mesh,
        scratch_types=[
            pltpu.VMEM((chunk,), jnp.float32),
            pltpu.VMEM((chunk,), jnp.int32),
            pltpu.VMEM((chunk,), jnp.float32),
            pltpu.VMEM((L,), jnp.float32),
            pltpu.VMEM((L,), jnp.float32),
            pltpu.SemaphoreType.DMA,
            pltpu.SemaphoreType.DMA,
            pltpu.SemaphoreType.DMA,
        ],
    )


def kernel(atomic_energy, atom_types, scales, shifts):
    n = atomic_energy.shape[0]
    e = atomic_energy.reshape(-1)
    t = atom_types.astype(jnp.int32)
    out = _build(n)(e, t, scales, shifts)
    return out.reshape(n, 1)


# speedup vs baseline: 1.0096x; 1.0069x over previous
import functools

import jax
import jax.numpy as jnp
from jax import lax
from jax.experimental import pallas as pl
from jax.experimental.pallas import tpu as pltpu
from jax.experimental.pallas import tpu_sc as plsc

NC = 1
NS = 16
NW = NC * NS
L = 16


def _gather16(table, idx):
    dnums = lax.GatherDimensionNumbers(
        offset_dims=(), collapsed_slice_dims=(0,), start_index_map=(0,))
    return lax.gather(table, idx[:, None], dnums, (1,),
                      mode=lax.GatherScatterMode.PROMISE_IN_BOUNDS)


def _scale_shift_body(n, energy_hbm, types_hbm, scales_hbm, shifts_hbm,
                      out_hbm, energy_v, types_v, out_v, scales_v, shifts_v,
                      sem_e, sem_t):
    wid = lax.axis_index("s") * NC + lax.axis_index("c")
    chunk = energy_v.shape[0]
    base = lax.min(wid * chunk, n - chunk)
    cp_e = pltpu.async_copy(energy_hbm.at[pl.ds(base, chunk)], energy_v, sem_e)
    cp_t = pltpu.async_copy(types_hbm.at[pl.ds(base, chunk)], types_v, sem_t)
    cp_s = pltpu.async_copy(scales_hbm, scales_v, sem_e)
    cp_b = pltpu.async_copy(shifts_hbm, shifts_v, sem_t)
    cp_s.wait()
    cp_b.wait()
    sv = scales_v[...]
    bv = shifts_v[...]
    cp_e.wait()
    cp_t.wait()

    @plsc.parallel_loop(0, chunk, L, unroll=4)
    def _(i):
        t = types_v[pl.ds(i, L)]
        s = _gather16(sv, t)
        b = _gather16(bv, t)
        e = energy_v[pl.ds(i, L)]
        out_v[pl.ds(i, L)] = s * e + b

    pltpu.sync_copy(out_v, out_hbm.at[pl.ds(base, chunk)])


@functools.lru_cache(maxsize=None)
def _build(n):
    chunk = -(-n // (NW * 2 * L)) * (2 * L)
    mesh = plsc.VectorSubcoreMesh(
        core_axis_name="c", subcore_axis_name="s",
        num_cores=NC, num_subcores=NS)
    return pl.kernel(
        functools.partial(_scale_shift_body, n),
        out_type=jax.ShapeDtypeStruct((n,), jnp.float32),
        mesh=mesh,
        scratch_types=[
            pltpu.VMEM((chunk,), jnp.float32),
            pltpu.VMEM((chunk,), jnp.int32),
            pltpu.VMEM((chunk,), jnp.float32),
            pltpu.VMEM((L,), jnp.float32),
            pltpu.VMEM((L,), jnp.float32),
            pltpu.SemaphoreType.DMA,
            pltpu.SemaphoreType.DMA,
        ],
    )


def kernel(atomic_energy, atom_types, scales, shifts):
    n = atomic_energy.shape[0]
    e = atomic_energy.reshape(-1)
    t = atom_types.astype(jnp.int32)
    out = _build(n)(e, t, scales, shifts)
    return out.reshape(n, 1)


# speedup vs baseline: 1.0110x; 1.0014x over previous
import functools

import jax
import jax.numpy as jnp
from jax import lax
from jax.experimental import pallas as pl
from jax.experimental.pallas import tpu as pltpu
from jax.experimental.pallas import tpu_sc as plsc

NC = 1
NS = 16
NW = NC * NS
L = 16


def _gather16(table, idx):
    dnums = lax.GatherDimensionNumbers(
        offset_dims=(), collapsed_slice_dims=(0,), start_index_map=(0,))
    return lax.gather(table, idx[:, None], dnums, (1,),
                      mode=lax.GatherScatterMode.PROMISE_IN_BOUNDS)


def _scale_shift_body(n, energy_hbm, types_hbm, scales_hbm, shifts_hbm,
                      out_hbm, energy_v, types_v, out_v, scales_v, shifts_v,
                      sem_e, sem_t, sem_tab):
    wid = lax.axis_index("s") * NC + lax.axis_index("c")
    chunk = energy_v.shape[0]
    base = lax.min(wid * chunk, n - chunk)
    cp_e = pltpu.async_copy(energy_hbm.at[pl.ds(base, chunk)], energy_v, sem_e)
    cp_t = pltpu.async_copy(types_hbm.at[pl.ds(base, chunk)], types_v, sem_t)
    cp_s = pltpu.async_copy(scales_hbm, scales_v, sem_tab)
    cp_b = pltpu.async_copy(shifts_hbm, shifts_v, sem_tab)
    cp_s.wait()
    cp_b.wait()
    sv = scales_v[...]
    bv = shifts_v[...]
    cp_e.wait()
    cp_t.wait()

    @plsc.parallel_loop(0, chunk, L, unroll=4)
    def _(i):
        t = types_v[pl.ds(i, L)]
        s = _gather16(sv, t)
        b = _gather16(bv, t)
        e = energy_v[pl.ds(i, L)]
        out_v[pl.ds(i, L)] = s * e + b

    pltpu.sync_copy(out_v, out_hbm.at[pl.ds(base, chunk)])


@functools.lru_cache(maxsize=None)
def _build(n):
    chunk = -(-n // (NW * 2 * L)) * (2 * L)
    mesh = plsc.VectorSubcoreMesh(
        core_axis_name="c", subcore_axis_name="s",
        num_cores=NC, num_subcores=NS)
    return pl.kernel(
        functools.partial(_scale_shift_body, n),
        out_type=jax.ShapeDtypeStruct((n,), jnp.float32),
        mesh=mesh,
        scratch_types=[
            pltpu.VMEM((chunk,), jnp.float32),
            pltpu.VMEM((chunk,), jnp.int32),
            pltpu.VMEM((chunk,), jnp.float32),
            pltpu.VMEM((L,), jnp.float32),
            pltpu.VMEM((L,), jnp.float32),
            pltpu.SemaphoreType.DMA,
            pltpu.SemaphoreType.DMA,
            pltpu.SemaphoreType.DMA,
        ],
    )


def kernel(atomic_energy, atom_types, scales, shifts):
    n = atomic_energy.shape[0]
    e = atomic_energy.reshape(-1)
    t = atom_types.astype(jnp.int32)
    out = _build(n)(e, t, scales, shifts)
    return out.reshape(n, 1)
